# Initial kernel scaffold; baseline (speedup 1.0000x reference)
import jax
import jax.numpy as jnp
from jax.experimental import pallas as pl


def kernel(x, context, ctx_w, ctx_b, q_w, q_b, k_w, k_b, v_w, v_b, proj_w, proj_b):
    raise NotImplementedError("write your pallas kernel here")


# speedup vs baseline: 3.7383x; 3.7383x over previous
import functools
import math

import jax
import jax.numpy as jnp
from jax import lax
from jax.experimental import pallas as pl
from jax.experimental.pallas import tpu as pltpu


def _prep_kernel(ce_ref, ctx_ref, wqk_ref, wctx_ref, bias_ref, aq_ref, ak_ref, *, C):
    y = jnp.dot(ce_ref[...], wqk_ref[...], preferred_element_type=jnp.float32)
    y = y + jnp.dot(ctx_ref[...], wctx_ref[...],
                    preferred_element_type=jnp.float32)
    y = y + bias_ref[...]
    aq_ref[...] = y[:, :C]
    ak_ref[...] = y[:, C:]


def _prep_terms(coords_enc, ctx_t, wqk, wctx, bias, *, C, tiles=2):
    plane = coords_enc.shape[0]
    C_qk = coords_enc.shape[1]
    tm = plane // tiles
    return pl.pallas_call(
        functools.partial(_prep_kernel, C=C),
        out_shape=(jax.ShapeDtypeStruct((plane, C), jnp.float32),
                   jax.ShapeDtypeStruct((plane, C), jnp.float32)),
        grid_spec=pltpu.PrefetchScalarGridSpec(
            num_scalar_prefetch=0,
            grid=(tiles,),
            in_specs=[
                pl.BlockSpec((tm, C_qk), lambda i: (i, 0)),
                pl.BlockSpec((tm, ctx_t.shape[1]), lambda i: (i, 0)),
                pl.BlockSpec((C_qk, 2 * C), lambda i: (0, 0)),
                pl.BlockSpec((wctx.shape[0], 2 * C), lambda i: (0, 0)),
                pl.BlockSpec((1, 2 * C), lambda i: (0, 0)),
            ],
            out_specs=[pl.BlockSpec((tm, C), lambda i: (i, 0)),
                       pl.BlockSpec((tm, C), lambda i: (i, 0))],
        ),
        compiler_params=pltpu.CompilerParams(dimension_semantics=("parallel",)),
    )(coords_enc, ctx_t, wqk, wctx, bias)


def _attn_kernel(tbl_ref, xq_ref, xkv_ref, aq_ref, ak_ref, wq_ref, wkv_ref,
                 vb_ref, pw_ref, pb_ref, o_ref, *, ws, Wp, C, nh, bn):
    b = pl.program_id(0)
    r = pl.program_id(1)
    g = b // bn
    hd = C // nh
    nwx = Wp // ws
    rows = ws * Wp
    L = ws * ws
    Lp = ((L + 7) // 8) * 8

    kvr = tbl_ref[b, r]

    xq = xq_ref[0].reshape(rows, C)
    xkv = xkv_ref[0].reshape(rows, C)

    q = jnp.dot(xq, wq_ref[...], preferred_element_type=jnp.float32)
    q = q + aq_ref[pl.ds(r * rows, rows), :]
    kv = jnp.dot(xkv, wkv_ref[...], preferred_element_type=jnp.float32)
    k = (kv[:, :C] + ak_ref[pl.ds(kvr * rows, rows), :]).reshape(ws, Wp, C)
    v = (kv[:, C:] + vb_ref[...]).reshape(ws, Wp, C)

    if nwx > 1:
        k_l = jnp.concatenate([k[:, ws:, :], k[:, Wp - ws:, :]], axis=1)
        k_r = jnp.concatenate([k[:, :ws, :], k[:, :Wp - ws, :]], axis=1)
        v_l = jnp.concatenate([v[:, ws:, :], v[:, Wp - ws:, :]], axis=1)
        v_r = jnp.concatenate([v[:, :ws, :], v[:, :Wp - ws, :]], axis=1)
        k = jnp.where(g == 2, k_l, jnp.where(g == 3, k_r, k))
        v = jnp.where(g == 2, v_l, jnp.where(g == 3, v_r, v))

    q = q.reshape(ws, Wp, C)

    rid = lax.broadcasted_iota(jnp.int32, (nh * Lp, C), 0) // Lp
    cid = lax.broadcasted_iota(jnp.int32, (nh * Lp, C), 1) // hd
    mask = rid == cid

    outs = []
    for wx in range(nwx):
        cs = slice(wx * ws, (wx + 1) * ws)
        qw = q[:, cs, :].reshape(L, C)
        kw = k[:, cs, :].reshape(L, C)
        vw = v[:, cs, :].reshape(L, C)
        qp = jnp.concatenate([qw, jnp.zeros((Lp - L, C), jnp.float32)], axis=0)
        qrep = jnp.where(mask, jnp.tile(qp, (nh, 1)), 0.0)
        s = lax.dot_general(qrep, kw, (((1,), (1,)), ((), ())),
                            preferred_element_type=jnp.float32)
        s = s - jnp.max(s, axis=-1, keepdims=True)
        p = jnp.exp(s)
        p = p * pl.reciprocal(jnp.sum(p, axis=-1, keepdims=True))
        pv = jnp.dot(p, vw, preferred_element_type=jnp.float32)
        pv = jnp.where(mask, pv, 0.0)
        ow = pv.reshape(nh, Lp, C).sum(axis=0)[:L]
        outs.append(ow.reshape(ws, ws, C))

    o_row = jnp.concatenate(outs, axis=1).reshape(rows, C)
    res = jnp.dot(o_row, pw_ref[...], preferred_element_type=jnp.float32)
    res = res + pb_ref[...]
    o_ref[0] = res.reshape(ws, Wp, C)


def _fused_attention(x4, kv_row, add_q, add_k, w_q, w_kv, v_b, proj_w, proj_b,
                     *, ws, nh, bn):
    B, Hp, Wp, C = x4.shape
    _h = Hp // ws
    plane = Hp * Wp

    q_map = lambda b, r, tbl: (b, r, 0, 0)
    kv_map = lambda b, r, tbl: (b, tbl[b, r], 0, 0)
    res_map = lambda b, r, tbl: (0, 0)

    out = pl.pallas_call(
        functools.partial(_attn_kernel, ws=ws, Wp=Wp, C=C, nh=nh, bn=bn),
        out_shape=jax.ShapeDtypeStruct((B, Hp, Wp, C), jnp.float32),
        grid_spec=pltpu.PrefetchScalarGridSpec(
            num_scalar_prefetch=1,
            grid=(B, _h),
            in_specs=[
                pl.BlockSpec((1, ws, Wp, C), q_map),
                pl.BlockSpec((1, ws, Wp, C), kv_map),
                pl.BlockSpec((plane, C), res_map),
                pl.BlockSpec((plane, C), res_map),
                pl.BlockSpec((C, C), res_map),
                pl.BlockSpec((C, 2 * C), res_map),
                pl.BlockSpec((1, C), res_map),
                pl.BlockSpec((C, C), res_map),
                pl.BlockSpec((1, C), res_map),
            ],
            out_specs=pl.BlockSpec((1, ws, Wp, C), q_map),
        ),
        compiler_params=pltpu.CompilerParams(
            dimension_semantics=("parallel", "parallel")),
    )(kv_row, x4, x4, add_q, add_k, w_q, w_kv, v_b, proj_w, proj_b)
    return out


def _sine_pos_enc(Hp, Wp, dim):
    freqs = jnp.arange(dim // 4, dtype=jnp.float32)
    yy, xx = jnp.meshgrid(jnp.arange(Hp, dtype=jnp.float32),
                          jnp.arange(Wp, dtype=jnp.float32), indexing="ij")
    wx = 3.14 * xx[..., None] * freqs * (1.0 / 200.0)
    wy = 3.14 * yy[..., None] * freqs * (1.0 / 200.0)
    return jnp.concatenate([jnp.sin(wx), jnp.cos(wx), jnp.sin(wy), jnp.cos(wy)],
                           axis=-1).reshape(Hp * Wp, dim)


def kernel(x, context, ctx_w, ctx_b, q_w, q_b, k_w, k_b, v_w, v_b,
           proj_w, proj_b):
    B, N, C = x.shape
    H = W = int(math.isqrt(N))
    ws = 7
    nh = 8
    vert_c_dim = q_w.shape[0] - C
    C_qk = C + vert_c_dim
    hd = C // nh
    scale = hd ** (-0.5)
    bn = B // 5
    _h = H // ws

    q_w = q_w * scale
    q_b = q_b * scale

    wqk = jnp.concatenate([q_w, k_w], axis=1)
    w_ctx2qk = ctx_w @ wqk[C:]
    bias = (jnp.concatenate([q_b, k_b], axis=0) + ctx_b @ wqk[C:]).reshape(1, 2 * C)

    coords_enc = _sine_pos_enc(H, W, C_qk)
    ctx_t = context.reshape(context.shape[1], H * W).T

    add_q, add_k = _prep_terms(coords_enc, ctx_t, wqk, w_ctx2qk, bias, C=C)

    wy = jnp.arange(_h, dtype=jnp.int32)
    row_up = jnp.minimum(wy + 1, _h - 1)
    row_dn = jnp.maximum(wy - 1, 0)
    gvec = (jnp.arange(B, dtype=jnp.int32) // bn)[:, None]
    kv_row = jnp.where(gvec == 0, row_up[None, :],
                       jnp.where(gvec == 1, row_dn[None, :],
                                 jnp.broadcast_to(wy[None, :], (B, _h)))).astype(jnp.int32)

    x4 = x.reshape(B, H, W, C)
    w_kv = jnp.concatenate([k_w[:C], v_w], axis=1)

    out = _fused_attention(x4, kv_row, add_q, add_k, q_w[:C], w_kv,
                           v_b.reshape(1, C), proj_w, proj_b.reshape(1, C),
                           ws=ws, nh=nh, bn=bn)
    return out.reshape(B, N, C)
